# Initial kernel scaffold; baseline (speedup 1.0000x reference)
import jax
import jax.numpy as jnp
from jax.experimental import pallas as pl


def kernel(x, length, table, W, b):
    raise NotImplementedError("write your pallas kernel here")


# speedup vs baseline: 3.0187x; 3.0187x over previous
import functools

import jax
import jax.numpy as jnp
from jax import lax
from jax.experimental import pallas as pl
from jax.experimental.pallas import tpu as pltpu
from jax.experimental.pallas import tpu_sc as plsc

_VOCAB = 1000000
_EMB = 64
_B = 16384
_L = 200

_TBLK = 25000

_NW = 32
_RPW = _B // _NW
_CH = 128
_NCH = _RPW // _CH
_IDXN = _CH * _L


def _tv_body(t_ref, w_ref, o_ref):
    o_ref[...] = jnp.dot(t_ref[...], w_ref[...],
                         preferred_element_type=jnp.float32)


def _table_times_w(table, W):
    return pl.pallas_call(
        _tv_body,
        grid=(_VOCAB // _TBLK,),
        in_specs=[
            pl.BlockSpec((_TBLK, _EMB), lambda i: (i, 0)),
            pl.BlockSpec((_EMB, 1), lambda i: (0, 0)),
        ],
        out_specs=pl.BlockSpec((_TBLK, 1), lambda i: (i, 0)),
        out_shape=jax.ShapeDtypeStruct((_VOCAB, 1), jnp.float32),
    )(table, W)


def _sc_pool(tv, xflat, length, b16):
    mesh = plsc.VectorSubcoreMesh(core_axis_name="c", subcore_axis_name="s")

    @functools.partial(
        pl.kernel,
        mesh=mesh,
        out_type=jax.ShapeDtypeStruct((_B,), jnp.float32),
        scratch_types=[
            pltpu.VMEM((_IDXN,), jnp.int32),
            pltpu.VMEM((_IDXN,), jnp.float32),
            pltpu.VMEM((_RPW,), jnp.float32),
            pltpu.VMEM((_RPW,), jnp.float32),
            pltpu.VMEM((16,), jnp.float32),
            pltpu.SemaphoreType.DMA,
        ],
    )
    def sck(tv_hbm, x_hbm, len_hbm, b_hbm, out_hbm,
            idx_v, val_v, len_v, out_v, b_v, sem):
        w = lax.axis_index("s") * 2 + lax.axis_index("c")
        row0 = w * _RPW
        pltpu.sync_copy(len_hbm.at[pl.ds(row0, _RPW)], len_v)
        pltpu.sync_copy(b_hbm, b_v)
        bias = b_v[...]
        for c in range(_NCH):
            pltpu.sync_copy(x_hbm.at[pl.ds((row0 + c * _CH) * _L, _IDXN)],
                            idx_v)
            pltpu.async_copy(tv_hbm.at[idx_v], val_v, sem).wait()
            for g in range(_CH // 16):
                goff = g * 16 * _L

                def jbody(j, acc, _goff=goff):
                    return acc + val_v[pl.ds(_goff + j * 16, 16)]

                acc = lax.fori_loop(0, _L, jbody,
                                    jnp.zeros((16,), jnp.float32))
                o16 = c * _CH + g * 16
                out_v[pl.ds(o16, 16)] = acc / len_v[pl.ds(o16, 16)] + bias
        pltpu.sync_copy(out_v, out_hbm.at[pl.ds(row0, _RPW)])

    return sck(tv, xflat, length, b16)


def kernel(x, length, table, W, b):
    tv = _table_times_w(table, W).reshape(_VOCAB)
    xflat = x.reshape(_B // 16, 16, _L).transpose(0, 2, 1).reshape(_B * _L)
    b16 = jnp.broadcast_to(b.astype(jnp.float32), (16,))
    rows = _sc_pool(tv, xflat, length, b16)
    return rows.reshape(_B, 1)


# speedup vs baseline: 9.2691x; 3.0705x over previous
import functools

import jax
import jax.numpy as jnp
from jax import lax
from jax.experimental import pallas as pl
from jax.experimental.pallas import tpu as pltpu
from jax.experimental.pallas import tpu_sc as plsc

_VOCAB = 1000000
_EMB = 64
_B = 16384
_L = 200

_TBLK = 24576

_NW = 32
_RPW = _B // _NW
_CH = 128
_NCH = _RPW // _CH
_IDXN = _CH * _L


def _tv_body(t_ref, w_ref, o_ref):
    o_ref[...] = jnp.sum(t_ref[...] * w_ref[...], axis=0)


def _table_times_w(table_t, W):
    return pl.pallas_call(
        _tv_body,
        grid=(pl.cdiv(_VOCAB, _TBLK),),
        in_specs=[
            pl.BlockSpec((_EMB, _TBLK), lambda i: (0, i)),
            pl.BlockSpec((_EMB, 1), lambda i: (0, 0)),
        ],
        out_specs=pl.BlockSpec((_TBLK,), lambda i: (i,)),
        out_shape=jax.ShapeDtypeStruct((_VOCAB,), jnp.float32),
    )(table_t, W)


def _sc_pool(tv, xflat, length, b16):
    mesh = plsc.VectorSubcoreMesh(core_axis_name="c", subcore_axis_name="s")

    @functools.partial(
        pl.kernel,
        mesh=mesh,
        out_type=jax.ShapeDtypeStruct((_B,), jnp.float32),
        scratch_types=[
            pltpu.VMEM((_IDXN,), jnp.int32),
            pltpu.VMEM((_IDXN,), jnp.float32),
            pltpu.VMEM((_RPW,), jnp.float32),
            pltpu.VMEM((_RPW,), jnp.float32),
            pltpu.VMEM((16,), jnp.float32),
            pltpu.SemaphoreType.DMA,
        ],
    )
    def sck(tv_hbm, x_hbm, len_hbm, b_hbm, out_hbm,
            idx_v, val_v, len_v, out_v, b_v, sem):
        w = lax.axis_index("s") * 2 + lax.axis_index("c")
        row0 = w * _RPW
        pltpu.sync_copy(len_hbm.at[pl.ds(row0, _RPW)], len_v)
        pltpu.sync_copy(b_hbm, b_v)
        bias = b_v[...]
        for c in range(_NCH):
            pltpu.sync_copy(x_hbm.at[pl.ds((row0 + c * _CH) * _L, _IDXN)],
                            idx_v)
            pltpu.async_copy(tv_hbm.at[idx_v], val_v, sem).wait()
            for g in range(_CH // 16):
                goff = g * 16 * _L

                def jbody(j, acc, _goff=goff):
                    return acc + val_v[pl.ds(_goff + j * 16, 16)]

                acc = lax.fori_loop(0, _L, jbody,
                                    jnp.zeros((16,), jnp.float32))
                o16 = c * _CH + g * 16
                out_v[pl.ds(o16, 16)] = acc / len_v[pl.ds(o16, 16)] + bias
        pltpu.sync_copy(out_v, out_hbm.at[pl.ds(row0, _RPW)])

    return sck(tv, xflat, length, b16)


def kernel(x, length, table, W, b):
    tv = _table_times_w(table.T, W)
    xflat = x.reshape(_B // 16, 16, _L).transpose(0, 2, 1).reshape(_B * _L)
    b16 = jnp.broadcast_to(b.astype(jnp.float32), (16,))
    rows = _sc_pool(tv, xflat, length, b16)
    return rows.reshape(_B, 1)
